# baseline (speedup 1.0000x reference)
import functools

import jax
import jax.numpy as jnp
from jax import lax
from jax.experimental import pallas as pl
from jax.experimental.pallas import tpu as pltpu
from jax.experimental.pallas import tpu_sc as plsc

_B, _HW, _C = 32, 1024, 768


def _sc_gather(table):
    mesh = plsc.VectorSubcoreMesh(core_axis_name="c", subcore_axis_name="s")
    info = plsc.get_sparse_core_info()
    num_cores = info.num_cores
    num_subcores = info.num_subcores
    n_workers = num_cores * num_subcores
    rows_per_sub = _HW // n_workers

    @functools.partial(
        pl.kernel,
        mesh=mesh,
        out_type=jax.ShapeDtypeStruct((_HW, _C), jnp.float32),
    )
    def k(table_hbm, emb_hbm):
        wid = lax.axis_index("s") * num_cores + lax.axis_index("c")
        row0 = wid * rows_per_sub
        pltpu.sync_copy(
            table_hbm.at[pl.ds(row0, rows_per_sub)],
            emb_hbm.at[pl.ds(row0, rows_per_sub)],
        )

    return k(table)


def _tc_broadcast(emb):
    def body(emb_ref, out_ref, sem):
        copies = [
            pltpu.make_async_copy(emb_ref, out_ref.at[i], sem)
            for i in range(_B)
        ]
        for c in copies:
            c.start()
        for c in copies:
            c.wait()

    return pl.pallas_call(
        body,
        in_specs=[pl.BlockSpec(memory_space=pltpu.VMEM)],
        out_specs=pl.BlockSpec(memory_space=pl.ANY),
        out_shape=jax.ShapeDtypeStruct((_B, _HW, _C), jnp.float32),
        scratch_shapes=[pltpu.SemaphoreType.DMA],
    )(emb)


def kernel(inputs, table):
    del inputs
    return _tc_broadcast(_sc_gather(table))


# speedup vs baseline: 2.6605x; 2.6605x over previous
import functools

import jax
import jax.numpy as jnp
from jax import lax
from jax.experimental import pallas as pl
from jax.experimental.pallas import tpu as pltpu
from jax.experimental.pallas import tpu_sc as plsc

_B, _HW, _C = 32, 1024, 768


def _sc_gather(table):
    info = plsc.get_sparse_core_info()
    num_cores = info.num_cores
    mesh = plsc.ScalarSubcoreMesh(axis_name="c", num_cores=num_cores)
    rows_per_core = _HW // num_cores

    @functools.partial(
        pl.kernel,
        mesh=mesh,
        out_type=jax.ShapeDtypeStruct((_HW, _C), jnp.float32),
        scratch_types=[pltpu.VMEM_SHARED((rows_per_core, _C), jnp.float32)],
    )
    def k(table_hbm, emb_hbm, spbuf):
        cid = lax.axis_index("c")
        row0 = cid * rows_per_core
        pltpu.sync_copy(table_hbm.at[pl.ds(row0, rows_per_core)], spbuf)
        pltpu.sync_copy(spbuf, emb_hbm.at[pl.ds(row0, rows_per_core)])

    return k(table)


def _tc_broadcast(emb):
    def body(emb_ref, out_ref, sem):
        copies = [
            pltpu.make_async_copy(emb_ref, out_ref.at[i], sem)
            for i in range(_B)
        ]
        for c in copies:
            c.start()
        for c in copies:
            c.wait()

    return pl.pallas_call(
        body,
        in_specs=[pl.BlockSpec(memory_space=pltpu.VMEM)],
        out_specs=pl.BlockSpec(memory_space=pl.ANY),
        out_shape=jax.ShapeDtypeStruct((_B, _HW, _C), jnp.float32),
        scratch_shapes=[pltpu.SemaphoreType.DMA],
    )(emb)


def kernel(inputs, table):
    del inputs
    return _tc_broadcast(_sc_gather(table))


# speedup vs baseline: 2.6714x; 1.0041x over previous
import functools

import jax
import jax.numpy as jnp
from jax import lax
from jax.experimental import pallas as pl
from jax.experimental.pallas import tpu as pltpu
from jax.experimental.pallas import tpu_sc as plsc

_B, _HW, _C = 32, 1024, 768
_SPLIT = 2
_HALF = _HW // _SPLIT


def _sc_gather(table):
    mesh = plsc.VectorSubcoreMesh(core_axis_name="c", subcore_axis_name="s")
    info = plsc.get_sparse_core_info()
    num_cores = info.num_cores
    num_subcores = info.num_subcores
    n_workers = num_cores * num_subcores
    rows_per_sub = _HW // n_workers

    @functools.partial(
        pl.kernel,
        mesh=mesh,
        out_type=jax.ShapeDtypeStruct((_HW, _C), jnp.float32),
        scratch_types=[pltpu.VMEM((rows_per_sub, _C), jnp.float32)],
    )
    def k(table_hbm, emb_hbm, buf):
        wid = lax.axis_index("s") * num_cores + lax.axis_index("c")
        row0 = wid * rows_per_sub
        pltpu.sync_copy(table_hbm.at[pl.ds(row0, rows_per_sub)], buf)
        pltpu.sync_copy(buf, emb_hbm.at[pl.ds(row0, rows_per_sub)])

    return k(table)


def _tc_broadcast(emb):
    def body(emb_ref, out_ref, sem0, sem1):
        sems = [sem0, sem1]
        copies = [
            pltpu.make_async_copy(
                emb_ref.at[pl.ds(j * _HALF, _HALF)],
                out_ref.at[i, pl.ds(j * _HALF, _HALF)],
                sems[(i * _SPLIT + j) % 2],
            )
            for i in range(_B)
            for j in range(_SPLIT)
        ]
        for c in copies:
            c.start()
        for c in copies:
            c.wait()

    return pl.pallas_call(
        body,
        in_specs=[pl.BlockSpec(memory_space=pltpu.VMEM)],
        out_specs=pl.BlockSpec(memory_space=pl.ANY),
        out_shape=jax.ShapeDtypeStruct((_B, _HW, _C), jnp.float32),
        scratch_shapes=[pltpu.SemaphoreType.DMA, pltpu.SemaphoreType.DMA],
    )(emb)


def kernel(inputs, table):
    del inputs
    return _tc_broadcast(_sc_gather(table))
